# baseline (speedup 1.0000x reference)
import functools

import jax
import jax.numpy as jnp
from jax import lax
from jax.experimental import pallas as pl
from jax.experimental.pallas import tpu as pltpu
from jax.experimental.pallas import tpu_sc as plsc

N_NODES = 10000
N_EDGES = 320000
D = 128
SLOPE = (1.0 / 8.0 + 1.0 / 3.0) / 2.0

NC = 2
NS = 16
NW = NC * NS
EDGES_PER_W = N_EDGES // NW

CHUNK = 40
BLK = 25
NBLK = 10
CPW = NBLK * BLK
EPW_PAD = CPW * CHUNK
PAD_E = EPW_PAD - EDGES_PER_W
ACC_ROWS = N_NODES + NS * 8

ROWS_MAIN = 640
ROWS_LAST = 400
ZR = 40
DEPTH = 5

MM_BLOCK = 1000


def _mm_body(x_ref, w_ref, o_ref):
    o_ref[...] = jnp.dot(x_ref[...], w_ref[...],
                         preferred_element_type=jnp.float32)


def _matmul(x, w):
    return pl.pallas_call(
        _mm_body,
        grid=(N_NODES // MM_BLOCK,),
        in_specs=[pl.BlockSpec((MM_BLOCK, D), lambda i: (i, 0)),
                  pl.BlockSpec((D, D), lambda i: (0, 0))],
        out_specs=pl.BlockSpec((MM_BLOCK, D), lambda i: (i, 0)),
        out_shape=jax.ShapeDtypeStruct((N_NODES, D), jnp.float32),
    )(x, w)


def _add_lrelu_mm_body(a_ref, b_ref, w_ref, o_ref):
    h = a_ref[...] + b_ref[...]
    h = jnp.maximum(h, SLOPE * h)
    o_ref[...] = jnp.dot(h, w_ref[...], preferred_element_type=jnp.float32)


def _add_lrelu_matmul(a, b, w):
    return pl.pallas_call(
        _add_lrelu_mm_body,
        grid=(N_NODES // MM_BLOCK,),
        in_specs=[pl.BlockSpec((MM_BLOCK, D), lambda i: (i, 0)),
                  pl.BlockSpec((MM_BLOCK, D), lambda i: (i, 0)),
                  pl.BlockSpec((D, D), lambda i: (0, 0))],
        out_specs=pl.BlockSpec((MM_BLOCK, D), lambda i: (i, 0)),
        out_shape=jax.ShapeDtypeStruct((N_NODES, D), jnp.float32),
    )(a, b, w)


def _add_lrelu_body(a_ref, b_ref, o_ref):
    h = a_ref[...] + b_ref[...]
    o_ref[...] = jnp.maximum(h, SLOPE * h)


def _add_lrelu(a, b):
    return pl.pallas_call(
        _add_lrelu_body,
        grid=(N_NODES // MM_BLOCK,),
        in_specs=[pl.BlockSpec((MM_BLOCK, D), lambda i: (i, 0)),
                  pl.BlockSpec((MM_BLOCK, D), lambda i: (i, 0))],
        out_specs=pl.BlockSpec((MM_BLOCK, D), lambda i: (i, 0)),
        out_shape=jax.ShapeDtypeStruct((N_NODES, D), jnp.float32),
    )(a, b)


def _seg_sum_sc(h, src, dst):
    mesh = plsc.VectorSubcoreMesh(core_axis_name="c", subcore_axis_name="s")

    @functools.partial(
        pl.kernel,
        mesh=mesh,
        out_type=[jax.ShapeDtypeStruct((N_NODES, D), jnp.float32),
                  jax.ShapeDtypeStruct((N_NODES, D), jnp.float32)],
        scratch_types=[
            pltpu.VMEM((BLK, CHUNK), jnp.int32),
            pltpu.VMEM((BLK, CHUNK), jnp.int32),
            pltpu.VMEM((BLK, CHUNK), jnp.int32),
            pltpu.VMEM((BLK, CHUNK), jnp.int32),
            pltpu.VMEM((CHUNK, D), jnp.float32),
            pltpu.VMEM((CHUNK, D), jnp.float32),
            pltpu.VMEM((CHUNK, D), jnp.float32),
            pltpu.VMEM((CHUNK, D), jnp.float32),
            pltpu.VMEM((CHUNK, D), jnp.float32),
            pltpu.VMEM_SHARED((ACC_ROWS, D), jnp.float32),
            pltpu.SemaphoreType.DMA,
            pltpu.SemaphoreType.DMA,
            pltpu.SemaphoreType.DMA,
            pltpu.SemaphoreType.DMA,
            pltpu.SemaphoreType.DMA,
            pltpu.SemaphoreType.DMA,
            pltpu.SemaphoreType.DMA,
            pltpu.SemaphoreType.DMA,
            pltpu.SemaphoreType.DMA,
        ],
    )
    def k(h_hbm, src_hbm, dst_hbm, out0_hbm, out1_hbm,
          sblk0, sblk1, dblk0, dblk1, rows0, rows1, rows2, rows3, rows4, acc,
          gsem0, gsem1, gsem2, gsem3, gsem4, ssem0, ssem1, dsem0, dsem1):
        c = lax.axis_index("c")
        s = lax.axis_index("s")
        wid = c * NS + s
        sblk = (sblk0, sblk1)
        dblk = (dblk0, dblk1)
        rows = (rows0, rows1, rows2, rows3, rows4)
        gsem = (gsem0, gsem1, gsem2, gsem3, gsem4)
        ssem = (ssem0, ssem1)
        dsem = (dsem0, dsem1)

        @pl.loop(0, ZR)
        def _(r):
            @pl.loop(0, D // 16)
            def _(j):
                rows0[r, pl.ds(j * 16, 16)] = jnp.zeros((16,), jnp.float32)

        @pl.loop(0, ROWS_MAIN // ZR)
        def _(z):
            @pl.when(jnp.logical_or(s < NS - 1, z < ROWS_LAST // ZR))
            def _():
                pltpu.sync_copy(
                    rows0.at[pl.ds(0, ZR)],
                    acc.at[pl.ds(s * ROWS_MAIN + z * ZR, ZR)])

        pltpu.sync_copy(rows0.at[pl.ds(0, 8)],
                        acc.at[pl.ds(N_NODES + s * 8, 8)])

        plsc.subcore_barrier()

        pltpu.sync_copy(src_hbm.at[wid * NBLK], sblk0)
        pltpu.sync_copy(dst_hbm.at[wid * NBLK], dblk0)
        for b in range(DEPTH):
            pltpu.make_async_copy(h_hbm.at[sblk0.at[b]], rows[b],
                                  gsem[b]).start()

        @pl.loop(0, NBLK, step=2)
        def _(g0):
            for gg in range(2):
                g = g0 + gg
                for i in range(BLK):
                    t = g * BLK + i
                    b = (gg * BLK + i) % DEPTH

                    if i == 0:
                        @pl.when(g + 1 < NBLK)
                        def _():
                            pltpu.make_async_copy(
                                src_hbm.at[wid * NBLK + g + 1],
                                sblk[1 - gg], ssem[1 - gg]).start()
                            pltpu.make_async_copy(
                                dst_hbm.at[wid * NBLK + g + 1],
                                dblk[1 - gg], dsem[1 - gg]).start()

                    if i == BLK - DEPTH - 1:
                        @pl.when(g + 1 < NBLK)
                        def _():
                            pltpu.make_async_copy(
                                src_hbm.at[wid * NBLK + g + 1],
                                sblk[1 - gg], ssem[1 - gg]).wait()
                            pltpu.make_async_copy(
                                dst_hbm.at[wid * NBLK + g + 1],
                                dblk[1 - gg], dsem[1 - gg]).wait()

                    pltpu.make_async_copy(
                        h_hbm.at[sblk[gg].at[i]], rows[b], gsem[b]).wait()
                    pltpu.sync_copy(rows[b], acc.at[dblk[gg].at[i]], add=True)

                    @pl.when(t + DEPTH < CPW)
                    def _():
                        if i + DEPTH < BLK:
                            pltpu.make_async_copy(
                                h_hbm.at[sblk[gg].at[i + DEPTH]],
                                rows[b], gsem[b]).start()
                        else:
                            pltpu.make_async_copy(
                                h_hbm.at[sblk[1 - gg].at[i + DEPTH - BLK]],
                                rows[b], gsem[b]).start()

        plsc.subcore_barrier()

        rbase = s * ROWS_MAIN

        @pl.when(c == 0)
        def _():
            pltpu.sync_copy(acc.at[pl.ds(rbase, ROWS_LAST)],
                            out0_hbm.at[pl.ds(rbase, ROWS_LAST)])

            @pl.when(s < NS - 1)
            def _():
                pltpu.sync_copy(
                    acc.at[pl.ds(rbase + ROWS_LAST, ROWS_MAIN - ROWS_LAST)],
                    out0_hbm.at[pl.ds(rbase + ROWS_LAST, ROWS_MAIN - ROWS_LAST)])

        @pl.when(c == 1)
        def _():
            pltpu.sync_copy(acc.at[pl.ds(rbase, ROWS_LAST)],
                            out1_hbm.at[pl.ds(rbase, ROWS_LAST)])

            @pl.when(s < NS - 1)
            def _():
                pltpu.sync_copy(
                    acc.at[pl.ds(rbase + ROWS_LAST, ROWS_MAIN - ROWS_LAST)],
                    out1_hbm.at[pl.ds(rbase + ROWS_LAST, ROWS_MAIN - ROWS_LAST)])

    return k(h, src, dst)


def _prep_edges(edge_index):
    ei = edge_index.astype(jnp.int32)
    src = ei[0].reshape(NW, EDGES_PER_W)
    dst = ei[1].reshape(NW, EDGES_PER_W)
    pad_src = jnp.zeros((NW, PAD_E), jnp.int32)
    sink_base = N_NODES + (jnp.arange(NW, dtype=jnp.int32) % NS) * 8
    pad_dst = sink_base[:, None] + (jnp.arange(PAD_E, dtype=jnp.int32) % 8)
    src = jnp.concatenate([src, pad_src], axis=1).reshape(NW * NBLK, BLK, CHUNK)
    dst = jnp.concatenate([dst, pad_dst], axis=1).reshape(NW * NBLK, BLK, CHUNK)
    return src, dst


def kernel(feat, edge_index, W1, W2):
    src, dst = _prep_edges(edge_index)

    h1 = _matmul(feat, W1)
    a0, a1 = _seg_sum_sc(h1, src, dst)
    h2 = _add_lrelu_matmul(a0, a1, W2)
    b0, b1 = _seg_sum_sc(h2, src, dst)
    return _add_lrelu(b0, b1)


# speedup vs baseline: 1.0132x; 1.0132x over previous
import functools

import jax
import jax.numpy as jnp
from jax import lax
from jax.experimental import pallas as pl
from jax.experimental.pallas import tpu as pltpu
from jax.experimental.pallas import tpu_sc as plsc

N_NODES = 10000
N_EDGES = 320000
D = 128
SLOPE = (1.0 / 8.0 + 1.0 / 3.0) / 2.0

NC = 2
NS = 16
NW = NC * NS
EDGES_PER_W = N_EDGES // NW

CHUNK = 50
BLK = 20
NBLK = 10
CPW = NBLK * BLK
EPW_PAD = CPW * CHUNK
PAD_E = EPW_PAD - EDGES_PER_W
ACC_ROWS = N_NODES + NS * 8

ROWS_MAIN = 640
ROWS_LAST = 400
ZR = 40
DEPTH = 5

MM_BLOCK = 1000


def _mm_body(x_ref, w_ref, o_ref):
    o_ref[...] = jnp.dot(x_ref[...], w_ref[...],
                         preferred_element_type=jnp.float32)


def _matmul(x, w):
    return pl.pallas_call(
        _mm_body,
        grid=(N_NODES // MM_BLOCK,),
        in_specs=[pl.BlockSpec((MM_BLOCK, D), lambda i: (i, 0)),
                  pl.BlockSpec((D, D), lambda i: (0, 0))],
        out_specs=pl.BlockSpec((MM_BLOCK, D), lambda i: (i, 0)),
        out_shape=jax.ShapeDtypeStruct((N_NODES, D), jnp.float32),
    )(x, w)


def _add_lrelu_mm_body(a_ref, b_ref, w_ref, o_ref):
    h = a_ref[...] + b_ref[...]
    h = jnp.maximum(h, SLOPE * h)
    o_ref[...] = jnp.dot(h, w_ref[...], preferred_element_type=jnp.float32)


def _add_lrelu_matmul(a, b, w):
    return pl.pallas_call(
        _add_lrelu_mm_body,
        grid=(N_NODES // MM_BLOCK,),
        in_specs=[pl.BlockSpec((MM_BLOCK, D), lambda i: (i, 0)),
                  pl.BlockSpec((MM_BLOCK, D), lambda i: (i, 0)),
                  pl.BlockSpec((D, D), lambda i: (0, 0))],
        out_specs=pl.BlockSpec((MM_BLOCK, D), lambda i: (i, 0)),
        out_shape=jax.ShapeDtypeStruct((N_NODES, D), jnp.float32),
    )(a, b, w)


def _add_lrelu_body(a_ref, b_ref, o_ref):
    h = a_ref[...] + b_ref[...]
    o_ref[...] = jnp.maximum(h, SLOPE * h)


def _add_lrelu(a, b):
    return pl.pallas_call(
        _add_lrelu_body,
        grid=(N_NODES // MM_BLOCK,),
        in_specs=[pl.BlockSpec((MM_BLOCK, D), lambda i: (i, 0)),
                  pl.BlockSpec((MM_BLOCK, D), lambda i: (i, 0))],
        out_specs=pl.BlockSpec((MM_BLOCK, D), lambda i: (i, 0)),
        out_shape=jax.ShapeDtypeStruct((N_NODES, D), jnp.float32),
    )(a, b)


def _seg_sum_sc(h, src, dst):
    mesh = plsc.VectorSubcoreMesh(core_axis_name="c", subcore_axis_name="s")

    @functools.partial(
        pl.kernel,
        mesh=mesh,
        out_type=[jax.ShapeDtypeStruct((N_NODES, D), jnp.float32),
                  jax.ShapeDtypeStruct((N_NODES, D), jnp.float32)],
        scratch_types=[
            pltpu.VMEM((BLK, CHUNK), jnp.int32),
            pltpu.VMEM((BLK, CHUNK), jnp.int32),
            pltpu.VMEM((BLK, CHUNK), jnp.int32),
            pltpu.VMEM((BLK, CHUNK), jnp.int32),
            pltpu.VMEM((CHUNK, D), jnp.float32),
            pltpu.VMEM((CHUNK, D), jnp.float32),
            pltpu.VMEM((CHUNK, D), jnp.float32),
            pltpu.VMEM((CHUNK, D), jnp.float32),
            pltpu.VMEM((CHUNK, D), jnp.float32),
            pltpu.VMEM_SHARED((ACC_ROWS, D), jnp.float32),
            pltpu.SemaphoreType.DMA,
            pltpu.SemaphoreType.DMA,
            pltpu.SemaphoreType.DMA,
            pltpu.SemaphoreType.DMA,
            pltpu.SemaphoreType.DMA,
            pltpu.SemaphoreType.DMA,
            pltpu.SemaphoreType.DMA,
            pltpu.SemaphoreType.DMA,
            pltpu.SemaphoreType.DMA,
        ],
    )
    def k(h_hbm, src_hbm, dst_hbm, out0_hbm, out1_hbm,
          sblk0, sblk1, dblk0, dblk1, rows0, rows1, rows2, rows3, rows4, acc,
          gsem0, gsem1, gsem2, gsem3, gsem4, ssem0, ssem1, dsem0, dsem1):
        c = lax.axis_index("c")
        s = lax.axis_index("s")
        wid = c * NS + s
        sblk = (sblk0, sblk1)
        dblk = (dblk0, dblk1)
        rows = (rows0, rows1, rows2, rows3, rows4)
        gsem = (gsem0, gsem1, gsem2, gsem3, gsem4)
        ssem = (ssem0, ssem1)
        dsem = (dsem0, dsem1)

        @pl.loop(0, ZR)
        def _(r):
            @pl.loop(0, D // 16)
            def _(j):
                rows0[r, pl.ds(j * 16, 16)] = jnp.zeros((16,), jnp.float32)

        @pl.loop(0, ROWS_MAIN // ZR)
        def _(z):
            @pl.when(jnp.logical_or(s < NS - 1, z < ROWS_LAST // ZR))
            def _():
                pltpu.sync_copy(
                    rows0.at[pl.ds(0, ZR)],
                    acc.at[pl.ds(s * ROWS_MAIN + z * ZR, ZR)])

        pltpu.sync_copy(rows0.at[pl.ds(0, 8)],
                        acc.at[pl.ds(N_NODES + s * 8, 8)])

        plsc.subcore_barrier()

        pltpu.sync_copy(src_hbm.at[wid * NBLK], sblk0)
        pltpu.sync_copy(dst_hbm.at[wid * NBLK], dblk0)
        for b in range(DEPTH):
            pltpu.make_async_copy(h_hbm.at[sblk0.at[b]], rows[b],
                                  gsem[b]).start()

        @pl.loop(0, NBLK, step=2)
        def _(g0):
            for gg in range(2):
                g = g0 + gg
                for i in range(BLK):
                    t = g * BLK + i
                    b = (gg * BLK + i) % DEPTH

                    if i == 0:
                        @pl.when(g + 1 < NBLK)
                        def _():
                            pltpu.make_async_copy(
                                src_hbm.at[wid * NBLK + g + 1],
                                sblk[1 - gg], ssem[1 - gg]).start()
                            pltpu.make_async_copy(
                                dst_hbm.at[wid * NBLK + g + 1],
                                dblk[1 - gg], dsem[1 - gg]).start()

                    if i == BLK - DEPTH - 1:
                        @pl.when(g + 1 < NBLK)
                        def _():
                            pltpu.make_async_copy(
                                src_hbm.at[wid * NBLK + g + 1],
                                sblk[1 - gg], ssem[1 - gg]).wait()
                            pltpu.make_async_copy(
                                dst_hbm.at[wid * NBLK + g + 1],
                                dblk[1 - gg], dsem[1 - gg]).wait()

                    pltpu.make_async_copy(
                        h_hbm.at[sblk[gg].at[i]], rows[b], gsem[b]).wait()
                    pltpu.sync_copy(rows[b], acc.at[dblk[gg].at[i]], add=True)

                    @pl.when(t + DEPTH < CPW)
                    def _():
                        if i + DEPTH < BLK:
                            pltpu.make_async_copy(
                                h_hbm.at[sblk[gg].at[i + DEPTH]],
                                rows[b], gsem[b]).start()
                        else:
                            pltpu.make_async_copy(
                                h_hbm.at[sblk[1 - gg].at[i + DEPTH - BLK]],
                                rows[b], gsem[b]).start()

        plsc.subcore_barrier()

        rbase = s * ROWS_MAIN

        @pl.when(c == 0)
        def _():
            pltpu.sync_copy(acc.at[pl.ds(rbase, ROWS_LAST)],
                            out0_hbm.at[pl.ds(rbase, ROWS_LAST)])

            @pl.when(s < NS - 1)
            def _():
                pltpu.sync_copy(
                    acc.at[pl.ds(rbase + ROWS_LAST, ROWS_MAIN - ROWS_LAST)],
                    out0_hbm.at[pl.ds(rbase + ROWS_LAST, ROWS_MAIN - ROWS_LAST)])

        @pl.when(c == 1)
        def _():
            pltpu.sync_copy(acc.at[pl.ds(rbase, ROWS_LAST)],
                            out1_hbm.at[pl.ds(rbase, ROWS_LAST)])

            @pl.when(s < NS - 1)
            def _():
                pltpu.sync_copy(
                    acc.at[pl.ds(rbase + ROWS_LAST, ROWS_MAIN - ROWS_LAST)],
                    out1_hbm.at[pl.ds(rbase + ROWS_LAST, ROWS_MAIN - ROWS_LAST)])

    return k(h, src, dst)


def _prep_edges(edge_index):
    ei = edge_index.astype(jnp.int32)
    src = ei[0].reshape(NW, EDGES_PER_W)
    dst = ei[1].reshape(NW, EDGES_PER_W)
    pad_src = jnp.zeros((NW, PAD_E), jnp.int32)
    sink_base = N_NODES + (jnp.arange(NW, dtype=jnp.int32) % NS) * 8
    pad_dst = sink_base[:, None] + (jnp.arange(PAD_E, dtype=jnp.int32) % 8)
    src = jnp.concatenate([src, pad_src], axis=1).reshape(NW * NBLK, BLK, CHUNK)
    dst = jnp.concatenate([dst, pad_dst], axis=1).reshape(NW * NBLK, BLK, CHUNK)
    return src, dst


def kernel(feat, edge_index, W1, W2):
    src, dst = _prep_edges(edge_index)

    h1 = _matmul(feat, W1)
    a0, a1 = _seg_sum_sc(h1, src, dst)
    h2 = _add_lrelu_matmul(a0, a1, W2)
    b0, b1 = _seg_sum_sc(h2, src, dst)
    return _add_lrelu(b0, b1)


# speedup vs baseline: 1.0431x; 1.0295x over previous
import functools

import jax
import jax.numpy as jnp
from jax import lax
from jax.experimental import pallas as pl
from jax.experimental.pallas import tpu as pltpu
from jax.experimental.pallas import tpu_sc as plsc

N_NODES = 10000
N_EDGES = 320000
D = 128
SLOPE = (1.0 / 8.0 + 1.0 / 3.0) / 2.0

NC = 2
NS = 16
NW = NC * NS
EDGES_PER_W = N_EDGES // NW

CHUNK = 50
BLK = 20
NBLK = 10
CPW = NBLK * BLK
EPW_PAD = CPW * CHUNK
PAD_E = EPW_PAD - EDGES_PER_W
ACC_ROWS = N_NODES + NS * 8

ROWS_MAIN = 640
ROWS_LAST = 400
ZR = 40
DEPTH = 5

MM_BLOCK = 2000


def _mm_body(x_ref, w_ref, o_ref):
    o_ref[...] = jnp.dot(x_ref[...], w_ref[...],
                         preferred_element_type=jnp.float32)


def _matmul(x, w):
    return pl.pallas_call(
        _mm_body,
        grid=(N_NODES // MM_BLOCK,),
        in_specs=[pl.BlockSpec((MM_BLOCK, D), lambda i: (i, 0)),
                  pl.BlockSpec((D, D), lambda i: (0, 0))],
        out_specs=pl.BlockSpec((MM_BLOCK, D), lambda i: (i, 0)),
        out_shape=jax.ShapeDtypeStruct((N_NODES, D), jnp.float32),
    )(x, w)


def _add_lrelu_mm_body(a_ref, b_ref, w_ref, o_ref):
    h = a_ref[...] + b_ref[...]
    h = jnp.maximum(h, SLOPE * h)
    o_ref[...] = jnp.dot(h, w_ref[...], preferred_element_type=jnp.float32)


def _add_lrelu_matmul(a, b, w):
    return pl.pallas_call(
        _add_lrelu_mm_body,
        grid=(N_NODES // MM_BLOCK,),
        in_specs=[pl.BlockSpec((MM_BLOCK, D), lambda i: (i, 0)),
                  pl.BlockSpec((MM_BLOCK, D), lambda i: (i, 0)),
                  pl.BlockSpec((D, D), lambda i: (0, 0))],
        out_specs=pl.BlockSpec((MM_BLOCK, D), lambda i: (i, 0)),
        out_shape=jax.ShapeDtypeStruct((N_NODES, D), jnp.float32),
    )(a, b, w)


def _add_lrelu_body(a_ref, b_ref, o_ref):
    h = a_ref[...] + b_ref[...]
    o_ref[...] = jnp.maximum(h, SLOPE * h)


def _add_lrelu(a, b):
    return pl.pallas_call(
        _add_lrelu_body,
        grid=(N_NODES // MM_BLOCK,),
        in_specs=[pl.BlockSpec((MM_BLOCK, D), lambda i: (i, 0)),
                  pl.BlockSpec((MM_BLOCK, D), lambda i: (i, 0))],
        out_specs=pl.BlockSpec((MM_BLOCK, D), lambda i: (i, 0)),
        out_shape=jax.ShapeDtypeStruct((N_NODES, D), jnp.float32),
    )(a, b)


def _seg_sum_sc(h, src, dst):
    mesh = plsc.VectorSubcoreMesh(core_axis_name="c", subcore_axis_name="s")

    @functools.partial(
        pl.kernel,
        mesh=mesh,
        out_type=[jax.ShapeDtypeStruct((N_NODES, D), jnp.float32),
                  jax.ShapeDtypeStruct((N_NODES, D), jnp.float32)],
        scratch_types=[
            pltpu.VMEM((BLK, CHUNK), jnp.int32),
            pltpu.VMEM((BLK, CHUNK), jnp.int32),
            pltpu.VMEM((BLK, CHUNK), jnp.int32),
            pltpu.VMEM((BLK, CHUNK), jnp.int32),
            pltpu.VMEM((CHUNK, D), jnp.float32),
            pltpu.VMEM((CHUNK, D), jnp.float32),
            pltpu.VMEM((CHUNK, D), jnp.float32),
            pltpu.VMEM((CHUNK, D), jnp.float32),
            pltpu.VMEM((CHUNK, D), jnp.float32),
            pltpu.VMEM_SHARED((ACC_ROWS, D), jnp.float32),
            pltpu.SemaphoreType.DMA,
            pltpu.SemaphoreType.DMA,
            pltpu.SemaphoreType.DMA,
            pltpu.SemaphoreType.DMA,
            pltpu.SemaphoreType.DMA,
            pltpu.SemaphoreType.DMA,
            pltpu.SemaphoreType.DMA,
            pltpu.SemaphoreType.DMA,
            pltpu.SemaphoreType.DMA,
        ],
    )
    def k(h_hbm, src_hbm, dst_hbm, out0_hbm, out1_hbm,
          sblk0, sblk1, dblk0, dblk1, rows0, rows1, rows2, rows3, rows4, acc,
          gsem0, gsem1, gsem2, gsem3, gsem4, ssem0, ssem1, dsem0, dsem1):
        c = lax.axis_index("c")
        s = lax.axis_index("s")
        wid = c * NS + s
        sblk = (sblk0, sblk1)
        dblk = (dblk0, dblk1)
        rows = (rows0, rows1, rows2, rows3, rows4)
        gsem = (gsem0, gsem1, gsem2, gsem3, gsem4)
        ssem = (ssem0, ssem1)
        dsem = (dsem0, dsem1)

        @pl.loop(0, ZR)
        def _(r):
            @pl.loop(0, D // 16)
            def _(j):
                rows0[r, pl.ds(j * 16, 16)] = jnp.zeros((16,), jnp.float32)

        @pl.loop(0, ROWS_MAIN // ZR)
        def _(z):
            @pl.when(jnp.logical_or(s < NS - 1, z < ROWS_LAST // ZR))
            def _():
                pltpu.sync_copy(
                    rows0.at[pl.ds(0, ZR)],
                    acc.at[pl.ds(s * ROWS_MAIN + z * ZR, ZR)])

        pltpu.sync_copy(rows0.at[pl.ds(0, 8)],
                        acc.at[pl.ds(N_NODES + s * 8, 8)])

        plsc.subcore_barrier()

        pltpu.sync_copy(src_hbm.at[wid * NBLK], sblk0)
        pltpu.sync_copy(dst_hbm.at[wid * NBLK], dblk0)
        for b in range(DEPTH):
            pltpu.make_async_copy(h_hbm.at[sblk0.at[b]], rows[b],
                                  gsem[b]).start()

        @pl.loop(0, NBLK, step=2)
        def _(g0):
            for gg in range(2):
                g = g0 + gg
                for i in range(BLK):
                    t = g * BLK + i
                    b = (gg * BLK + i) % DEPTH

                    if i == 0:
                        @pl.when(g + 1 < NBLK)
                        def _():
                            pltpu.make_async_copy(
                                src_hbm.at[wid * NBLK + g + 1],
                                sblk[1 - gg], ssem[1 - gg]).start()
                            pltpu.make_async_copy(
                                dst_hbm.at[wid * NBLK + g + 1],
                                dblk[1 - gg], dsem[1 - gg]).start()

                    if i == BLK - DEPTH - 1:
                        @pl.when(g + 1 < NBLK)
                        def _():
                            pltpu.make_async_copy(
                                src_hbm.at[wid * NBLK + g + 1],
                                sblk[1 - gg], ssem[1 - gg]).wait()
                            pltpu.make_async_copy(
                                dst_hbm.at[wid * NBLK + g + 1],
                                dblk[1 - gg], dsem[1 - gg]).wait()

                    pltpu.make_async_copy(
                        h_hbm.at[sblk[gg].at[i]], rows[b], gsem[b]).wait()
                    pltpu.sync_copy(rows[b], acc.at[dblk[gg].at[i]], add=True)

                    @pl.when(t + DEPTH < CPW)
                    def _():
                        if i + DEPTH < BLK:
                            pltpu.make_async_copy(
                                h_hbm.at[sblk[gg].at[i + DEPTH]],
                                rows[b], gsem[b]).start()
                        else:
                            pltpu.make_async_copy(
                                h_hbm.at[sblk[1 - gg].at[i + DEPTH - BLK]],
                                rows[b], gsem[b]).start()

        plsc.subcore_barrier()

        rbase = s * ROWS_MAIN

        @pl.when(c == 0)
        def _():
            pltpu.sync_copy(acc.at[pl.ds(rbase, ROWS_LAST)],
                            out0_hbm.at[pl.ds(rbase, ROWS_LAST)])

            @pl.when(s < NS - 1)
            def _():
                pltpu.sync_copy(
                    acc.at[pl.ds(rbase + ROWS_LAST, ROWS_MAIN - ROWS_LAST)],
                    out0_hbm.at[pl.ds(rbase + ROWS_LAST, ROWS_MAIN - ROWS_LAST)])

        @pl.when(c == 1)
        def _():
            pltpu.sync_copy(acc.at[pl.ds(rbase, ROWS_LAST)],
                            out1_hbm.at[pl.ds(rbase, ROWS_LAST)])

            @pl.when(s < NS - 1)
            def _():
                pltpu.sync_copy(
                    acc.at[pl.ds(rbase + ROWS_LAST, ROWS_MAIN - ROWS_LAST)],
                    out1_hbm.at[pl.ds(rbase + ROWS_LAST, ROWS_MAIN - ROWS_LAST)])

    return k(h, src, dst)


def _prep_edges(edge_index):
    ei = edge_index.astype(jnp.int32)
    src = ei[0].reshape(NW, EDGES_PER_W)
    dst = ei[1].reshape(NW, EDGES_PER_W)
    pad_src = jnp.zeros((NW, PAD_E), jnp.int32)
    sink_base = N_NODES + (jnp.arange(NW, dtype=jnp.int32) % NS) * 8
    pad_dst = sink_base[:, None] + (jnp.arange(PAD_E, dtype=jnp.int32) % 8)
    src = jnp.concatenate([src, pad_src], axis=1).reshape(NW * NBLK, BLK, CHUNK)
    dst = jnp.concatenate([dst, pad_dst], axis=1).reshape(NW * NBLK, BLK, CHUNK)
    return src, dst


def kernel(feat, edge_index, W1, W2):
    src, dst = _prep_edges(edge_index)

    h1 = _matmul(feat, W1)
    a0, a1 = _seg_sum_sc(h1, src, dst)
    h2 = _add_lrelu_matmul(a0, a1, W2)
    b0, b1 = _seg_sum_sc(h2, src, dst)
    return _add_lrelu(b0, b1)
